# baseline (speedup 1.0000x reference)
import functools

import jax
import jax.numpy as jnp
from jax.experimental import pallas as pl

_BM = 1000


def _mm_kernel(a_ref, w_ref, b_ref, o_ref, *, relu):
    acc = jnp.dot(a_ref[...], w_ref[...], preferred_element_type=jnp.float32)
    acc = acc + b_ref[...]
    o_ref[...] = jnp.maximum(acc, 0.0) if relu else acc


def _matmul(a, w, b, relu=False):
    m, k = a.shape
    n = w.shape[1]
    return pl.pallas_call(
        functools.partial(_mm_kernel, relu=relu),
        grid=(m // _BM,),
        in_specs=[
            pl.BlockSpec((_BM, k), lambda i: (i, 0)),
            pl.BlockSpec((k, n), lambda i: (0, 0)),
            pl.BlockSpec((1, n), lambda i: (0, 0)),
        ],
        out_specs=pl.BlockSpec((_BM, n), lambda i: (i, 0)),
        out_shape=jax.ShapeDtypeStruct((m, n), jnp.float32),
    )(a, w, jnp.reshape(b, (1, n)))


def _stats_kernel(*refs, has_add, has_bias):
    z = refs[0][...]
    i = 1
    if has_add:
        z = z + refs[i][...]
        i += 1
    if has_bias:
        z = z + refs[i][...]
        i += 1
    o_ref = refs[i]
    s1 = jnp.sum(z, axis=0, keepdims=True)
    s2 = jnp.sum(z * z, axis=0, keepdims=True)
    upd = jnp.concatenate(
        [s1, s2, jnp.zeros((6, z.shape[1]), jnp.float32)], axis=0)

    @pl.when(pl.program_id(0) == 0)
    def _():
        o_ref[...] = jnp.zeros_like(o_ref)

    o_ref[...] += upd


def _stats(z, addend=None, bias=None):
    m, n = z.shape
    args = [z]
    specs = [pl.BlockSpec((_BM, n), lambda i: (i, 0))]
    if addend is not None:
        args.append(addend)
        specs.append(pl.BlockSpec((_BM, n), lambda i: (i, 0)))
    if bias is not None:
        args.append(jnp.reshape(bias, (1, n)))
        specs.append(pl.BlockSpec((1, n), lambda i: (0, 0)))
    return pl.pallas_call(
        functools.partial(_stats_kernel, has_add=addend is not None,
                          has_bias=bias is not None),
        grid=(m // _BM,),
        in_specs=specs,
        out_specs=pl.BlockSpec((8, n), lambda i: (0, 0)),
        out_shape=jax.ShapeDtypeStruct((8, n), jnp.float32),
    )(*args)


def _bn_apply_kernel(*refs, has_add, has_bias, has_res, has_post, relu, m):
    z = refs[0][...]
    i = 1
    if has_add:
        z = z + refs[i][...]
        i += 1
    if has_bias:
        z = z + refs[i][...]
        i += 1
    st = refs[i][...]
    g = refs[i + 1][...]
    bb = refs[i + 2][...]
    i += 3
    mean = st[0:1, :] / m
    var = st[1:2, :] / m - mean * mean
    inv = jax.lax.rsqrt(var + 1e-5)
    out = (z - mean) * inv * g + bb
    if relu:
        out = jnp.maximum(out, 0.0)
    if has_res:
        out = out + refs[i][...]
        i += 1
    if has_post:
        out = out + refs[i][...] * 0.1
        i += 1
    refs[i][...] = out


def _bn_apply(z, st, g, bb, addend=None, bias=None, residual=None,
              post=None, relu=True):
    m, n = z.shape
    args = [z]
    specs = [pl.BlockSpec((_BM, n), lambda i: (i, 0))]
    if addend is not None:
        args.append(addend)
        specs.append(pl.BlockSpec((_BM, n), lambda i: (i, 0)))
    if bias is not None:
        args.append(jnp.reshape(bias, (1, n)))
        specs.append(pl.BlockSpec((1, n), lambda i: (0, 0)))
    args += [st, jnp.reshape(g, (1, n)), jnp.reshape(bb, (1, n))]
    specs += [
        pl.BlockSpec((8, n), lambda i: (0, 0)),
        pl.BlockSpec((1, n), lambda i: (0, 0)),
        pl.BlockSpec((1, n), lambda i: (0, 0)),
    ]
    if residual is not None:
        args.append(residual)
        specs.append(pl.BlockSpec((_BM, n), lambda i: (i, 0)))
    if post is not None:
        args.append(jnp.reshape(post, (1, n)))
        specs.append(pl.BlockSpec((1, n), lambda i: (0, 0)))
    return pl.pallas_call(
        functools.partial(_bn_apply_kernel, has_add=addend is not None,
                          has_bias=bias is not None,
                          has_res=residual is not None,
                          has_post=post is not None, relu=relu, m=float(m)),
        grid=(m // _BM,),
        in_specs=specs,
        out_specs=pl.BlockSpec((_BM, n), lambda i: (i, 0)),
        out_shape=jax.ShapeDtypeStruct((m, n), jnp.float32),
    )(*args)


def kernel(x, edge_index, We1, be1, g_e, bb_e, We2, be2, Wg, bg, g1, bb1,
           Wa, att_src, att_dst, ba, g2, bb2, Wk, Wq, Wv, Ws, br, g3, bb3,
           gf, Wc1, bc1, Wc2, bc2):
    n = x.shape[0]
    heads, h_dim = att_src.shape
    src = edge_index[0]
    dst = edge_index[1]
    sl = jnp.arange(n, dtype=src.dtype)
    s2 = jnp.concatenate([src, sl])
    d2 = jnp.concatenate([dst, sl])

    din = x.shape[1]
    kpad = (-din) % 128
    xp = jnp.pad(x, ((0, 0), (0, kpad)))
    We1p = jnp.pad(We1, ((0, kpad), (0, 0)))
    z1 = _matmul(xp, We1p, be1)
    st1 = _stats(z1)
    h = _bn_apply(z1, st1, g_e, bb_e, relu=True)
    h = _matmul(h, We2, be2, relu=True)
    identity = h

    xw = _matmul(h, Wg, jnp.zeros((h_dim,), jnp.float32))
    deg = jax.ops.segment_sum(jnp.ones((d2.shape[0],), jnp.float32), d2,
                              num_segments=n)
    dinv = jax.lax.rsqrt(jnp.maximum(deg, 1e-12))
    msg = xw[s2] * (dinv[s2] * dinv[d2])[:, None]
    agg1 = jax.ops.segment_sum(msg, d2, num_segments=n)
    stg = _stats(agg1, bias=bg)
    h = _bn_apply(agg1, stg, g1, bb1, bias=bg, residual=identity, relu=True)

    xw_a = _matmul(h, Wa, jnp.zeros((heads * h_dim,), jnp.float32))
    att_mat = jnp.zeros((heads * h_dim, 128), jnp.float32)
    for hd in range(heads):
        att_mat = att_mat.at[hd * h_dim:(hd + 1) * h_dim, hd].set(att_src[hd])
        att_mat = att_mat.at[hd * h_dim:(hd + 1) * h_dim,
                             heads + hd].set(att_dst[hd])
    att_out = _matmul(xw_a, att_mat, jnp.zeros((128,), jnp.float32))
    asrc = att_out[:, :heads]
    adst = att_out[:, heads:2 * heads]
    xw3 = xw_a.reshape(n, heads, h_dim)
    e = jax.nn.leaky_relu(asrc[s2] + adst[d2], negative_slope=0.2)
    emax = jax.ops.segment_max(e, d2, num_segments=n)
    ee = jnp.exp(e - emax[d2])
    denom = jax.ops.segment_sum(ee, d2, num_segments=n)
    alpha = ee / (denom[d2] + 1e-16)
    agg2 = jax.ops.segment_sum(xw3[s2] * alpha[:, :, None], d2,
                               num_segments=n)
    agg2 = agg2.reshape(n, heads * h_dim)
    st2 = _stats(agg2, bias=ba)
    h = _bn_apply(agg2, st2, g2, bb2, bias=ba, relu=True)

    Wcat = jnp.concatenate([Wk, Wq, Wv, Ws], axis=1)
    kqvs = _matmul(h, Wcat, jnp.zeros((4 * h_dim,), jnp.float32))
    k = kqvs[:, 0:h_dim]
    q = kqvs[:, h_dim:2 * h_dim]
    v = kqvs[:, 2 * h_dim:3 * h_dim]
    s_out = kqvs[:, 3 * h_dim:4 * h_dim]
    eta = jax.nn.sigmoid(k[dst] + q[src])
    agg3 = jax.ops.segment_sum(eta * v[src], dst, num_segments=n)
    st3 = _stats(agg3, addend=s_out, bias=br)
    h = _bn_apply(agg3, st3, g3, bb3, addend=s_out, bias=br, post=gf,
                  relu=True)

    out = _matmul(h, Wc1, bc1, relu=True)
    npad = (-Wc2.shape[1]) % 128
    Wc2p = jnp.pad(Wc2, ((0, 0), (0, npad)))
    bc2p = jnp.pad(bc2, (0, npad))
    out = _matmul(out, Wc2p, bc2p)
    return out[:, :Wc2.shape[1]]
